# baseline (speedup 1.0000x reference)
import functools

import jax
import jax.numpy as jnp
from jax import lax
from jax.experimental import pallas as pl
from jax.experimental.pallas import tpu as pltpu
from jax.experimental.pallas import tpu_sc as plsc

V = 1000000
D = 64
HID = 600
OUT = 1000
B = 4096
TLEN = 20
SLEN = 200

NC = 2
NS = 16
NW = NC * NS
BPW = B // NW
GROUP = 2
NG = BPW // GROUP
NLANE = 16


def _pool_body(title_hbm, snip_hbm, ttab_hbm, stab_hbm, out_hbm,
               idx_t, idx_s, tbuf, sbuf, acc, sem):
    cid = lax.axis_index("c")
    sid = lax.axis_index("s")
    wid = sid * NC + cid

    pltpu.sync_copy(title_hbm.at[pl.ds(wid * (BPW * TLEN), BPW * TLEN)], idx_t)
    pltpu.sync_copy(snip_hbm.at[pl.ds(wid * (BPW * SLEN), BPW * SLEN)], idx_s)

    def group(g, carry):
        cp_t = pltpu.async_copy(
            ttab_hbm.at[idx_t.at[pl.ds(g * (GROUP * TLEN), GROUP * TLEN)]],
            tbuf, sem)
        cp_s = pltpu.async_copy(
            stab_hbm.at[idx_s.at[pl.ds(g * (GROUP * SLEN), GROUP * SLEN)]],
            sbuf, sem)
        cp_t.wait()
        cp_s.wait()

        for k in range(GROUP):
            row = g * GROUP + k

            def tbody(t, vs):
                base = k * TLEN + t
                return tuple(vs[c] + tbuf[base, pl.ds(c * NLANE, NLANE)]
                             for c in range(D // NLANE))

            tz = tuple(jnp.zeros((NLANE,), jnp.float32)
                       for _ in range(D // NLANE))
            tv = lax.fori_loop(0, TLEN, tbody, tz)
            for c in range(D // NLANE):
                acc[row, pl.ds(c * NLANE, NLANE)] = tv[c] * (1.0 / TLEN)

            def sbody(t, vs):
                base = k * SLEN + t
                return tuple(vs[c] + sbuf[base, pl.ds(c * NLANE, NLANE)]
                             for c in range(D // NLANE))

            sv = lax.fori_loop(0, SLEN, sbody, tz)
            for c in range(D // NLANE):
                acc[row, pl.ds(D + c * NLANE, NLANE)] = sv[c] * (1.0 / SLEN)
        return carry

    lax.fori_loop(0, NG, group, 0)

    pltpu.sync_copy(acc, out_hbm.at[pl.ds(wid * BPW, BPW)])


@functools.partial(jax.jit, static_argnums=())
def _pool(title_flat, snip_flat, ttab, stab):
    mesh = plsc.VectorSubcoreMesh(core_axis_name="c", subcore_axis_name="s")
    fn = pl.kernel(
        _pool_body,
        mesh=mesh,
        out_type=jax.ShapeDtypeStruct((B, 2 * D), jnp.float32),
        scratch_types=[
            pltpu.VMEM((BPW * TLEN,), jnp.int32),
            pltpu.VMEM((BPW * SLEN,), jnp.int32),
            pltpu.VMEM((GROUP * TLEN, D), jnp.float32),
            pltpu.VMEM((GROUP * SLEN, D), jnp.float32),
            pltpu.VMEM((BPW, 2 * D), jnp.float32),
            pltpu.SemaphoreType.DMA,
        ],
        compiler_params=pltpu.CompilerParams(use_tc_tiling_on_sc=False),
    )
    return fn(title_flat, snip_flat, ttab, stab)


def _mlp_body(x_ref, w1_ref, b1_ref, w2_ref, b2_ref, o_ref):
    h = jnp.dot(x_ref[...], w1_ref[...], preferred_element_type=jnp.float32)
    h = jnp.maximum(h + b1_ref[...], 0.0)
    o_ref[...] = (jnp.dot(h, w2_ref[...], preferred_element_type=jnp.float32)
                  + b2_ref[...])


def _mlp(x, W1, b1, W2, b2):
    TB = 512
    grid = (B // TB,)
    return pl.pallas_call(
        _mlp_body,
        grid=grid,
        in_specs=[
            pl.BlockSpec((TB, 2 * D), lambda i: (i, 0)),
            pl.BlockSpec((2 * D, HID), lambda i: (0, 0)),
            pl.BlockSpec((1, HID), lambda i: (0, 0)),
            pl.BlockSpec((HID, OUT), lambda i: (0, 0)),
            pl.BlockSpec((1, OUT), lambda i: (0, 0)),
        ],
        out_specs=pl.BlockSpec((TB, OUT), lambda i: (i, 0)),
        out_shape=jax.ShapeDtypeStruct((B, OUT), jnp.float32),
    )(x, W1, b1, W2, b2)


def kernel(title, snippet, title_table, snippet_table, W1, b1, W2, b2):
    title_flat = title.astype(jnp.int32).reshape(-1)
    snip_flat = snippet.astype(jnp.int32).reshape(-1)
    combined = _pool(title_flat, snip_flat, title_table, snippet_table)
    return _mlp(combined, W1, b1.reshape(1, HID), W2, b2.reshape(1, OUT))


# speedup vs baseline: 1.0719x; 1.0719x over previous
import functools

import jax
import jax.numpy as jnp
from jax import lax
from jax.experimental import pallas as pl
from jax.experimental.pallas import tpu as pltpu
from jax.experimental.pallas import tpu_sc as plsc

V = 1000000
D = 64
HID = 600
OUT = 1000
B = 4096
TLEN = 20
SLEN = 200

NC = 2
NS = 16
NW = NC * NS
BPW = B // NW
NLANE = 16
NV = D // NLANE


def _pool_body(title_hbm, snip_hbm, ttab_hbm, stab_hbm, out_hbm,
               idx_t, idx_s, tbufA, sbufA, tbufB, sbufB, acc, semA, semB):
    cid = lax.axis_index("c")
    sid = lax.axis_index("s")
    wid = sid * NC + cid
    base = wid * BPW

    pltpu.sync_copy(title_hbm.at[pl.ds(base, BPW)], idx_t)
    pltpu.sync_copy(snip_hbm.at[pl.ds(base, BPW)], idx_s)

    def fire(row, tbuf, sbuf, sem):
        pltpu.async_copy(ttab_hbm.at[idx_t.at[row]], tbuf, sem)
        pltpu.async_copy(stab_hbm.at[idx_s.at[row]], sbuf, sem)

    def drain(row, tbuf, sbuf, sem):
        pltpu.make_async_copy(ttab_hbm.at[idx_t.at[row]], tbuf, sem).wait()
        pltpu.make_async_copy(stab_hbm.at[idx_s.at[row]], sbuf, sem).wait()

    def accumulate(row, tbuf, sbuf):
        zeros = tuple(jnp.zeros((NLANE,), jnp.float32) for _ in range(NV))

        def tbody(t, vs):
            return tuple(
                vs[c] + tbuf[2 * t, pl.ds(c * NLANE, NLANE)]
                + tbuf[2 * t + 1, pl.ds(c * NLANE, NLANE)]
                for c in range(NV))

        tv = lax.fori_loop(0, TLEN // 2, tbody, zeros)
        for c in range(NV):
            acc[row, pl.ds(c * NLANE, NLANE)] = tv[c] * (1.0 / TLEN)

        def sbody(t, vs):
            return tuple(
                vs[c] + sbuf[2 * t, pl.ds(c * NLANE, NLANE)]
                + sbuf[2 * t + 1, pl.ds(c * NLANE, NLANE)]
                for c in range(NV))

        sv = lax.fori_loop(0, SLEN // 2, sbody, zeros)
        for c in range(NV):
            acc[row, pl.ds(D + c * NLANE, NLANE)] = sv[c] * (1.0 / SLEN)

    fire(0, tbufA, sbufA, semA)
    bufs = ((tbufA, sbufA, semA), (tbufB, sbufB, semB))

    def body(i, carry):
        for p in range(2):
            row = i * 2 + p
            tbuf, sbuf, sem = bufs[p]
            ntbuf, nsbuf, nsem = bufs[1 - p]

            @pl.when(row + 1 < BPW)
            def _():
                fire(row + 1, ntbuf, nsbuf, nsem)

            drain(row, tbuf, sbuf, sem)
            accumulate(row, tbuf, sbuf)
        return carry

    lax.fori_loop(0, BPW // 2, body, 0)

    pltpu.sync_copy(acc, out_hbm.at[pl.ds(base, BPW)])


def _pool(title, snippet, ttab, stab):
    mesh = plsc.VectorSubcoreMesh(core_axis_name="c", subcore_axis_name="s")
    fn = pl.kernel(
        _pool_body,
        mesh=mesh,
        out_type=jax.ShapeDtypeStruct((B, 2 * D), jnp.float32),
        scratch_types=[
            pltpu.VMEM((BPW, TLEN), jnp.int32),
            pltpu.VMEM((BPW, SLEN), jnp.int32),
            pltpu.VMEM((TLEN, D), jnp.float32),
            pltpu.VMEM((SLEN, D), jnp.float32),
            pltpu.VMEM((TLEN, D), jnp.float32),
            pltpu.VMEM((SLEN, D), jnp.float32),
            pltpu.VMEM((BPW, 2 * D), jnp.float32),
            pltpu.SemaphoreType.DMA,
            pltpu.SemaphoreType.DMA,
        ],
        compiler_params=pltpu.CompilerParams(use_tc_tiling_on_sc=False),
    )
    return fn(title, snippet, ttab, stab)


def _mlp_body(x_ref, w1_ref, b1_ref, w2_ref, b2_ref, o_ref):
    h = jnp.dot(x_ref[...], w1_ref[...], preferred_element_type=jnp.float32)
    h = jnp.maximum(h + b1_ref[...], 0.0)
    o_ref[...] = (jnp.dot(h, w2_ref[...], preferred_element_type=jnp.float32)
                  + b2_ref[...])


def _mlp(x, W1, b1, W2, b2):
    TB = 512
    grid = (B // TB,)
    return pl.pallas_call(
        _mlp_body,
        grid=grid,
        in_specs=[
            pl.BlockSpec((TB, 2 * D), lambda i: (i, 0)),
            pl.BlockSpec((2 * D, HID), lambda i: (0, 0)),
            pl.BlockSpec((1, HID), lambda i: (0, 0)),
            pl.BlockSpec((HID, OUT), lambda i: (0, 0)),
            pl.BlockSpec((1, OUT), lambda i: (0, 0)),
        ],
        out_specs=pl.BlockSpec((TB, OUT), lambda i: (i, 0)),
        out_shape=jax.ShapeDtypeStruct((B, OUT), jnp.float32),
    )(x, W1, b1, W2, b2)


def kernel(title, snippet, title_table, snippet_table, W1, b1, W2, b2):
    combined = _pool(title.astype(jnp.int32), snippet.astype(jnp.int32),
                     title_table, snippet_table)
    return _mlp(combined, W1, b1.reshape(1, HID), W2, b2.reshape(1, OUT))


# speedup vs baseline: 1.0912x; 1.0180x over previous
import jax
import jax.numpy as jnp
from jax import lax
from jax.experimental import pallas as pl
from jax.experimental.pallas import tpu as pltpu
from jax.experimental.pallas import tpu_sc as plsc

V = 1000000
D = 64
HID = 600
OUT = 1000
B = 4096
TLEN = 20
SLEN = 200

NC = 2
NS = 16
NW = NC * NS
BPW = B // NW
GROUP = 2
NG = BPW // GROUP
NLANE = 16
NV = D // NLANE


def _flatten_row(src2d, dst1d, row, dst_base, n, iota):
    full = n - n % NLANE
    for c in range(0, full, NLANE):
        vals = src2d[row, pl.ds(c, NLANE)]
        plsc.store_scatter(dst1d, [dst_base + c + iota], vals)
    if full < n:
        tail = n - NLANE
        rows = jnp.full((NLANE,), row, jnp.int32)
        vals = plsc.load_gather(src2d, [rows, tail + iota])
        plsc.store_scatter(dst1d, [dst_base + tail + iota], vals)


def _flatten_body(title_hbm, snip_hbm, tout_hbm, sout_hbm,
                  t2d, s2d, tfl, sfl):
    cid = lax.axis_index("c")
    sid = lax.axis_index("s")
    wid = sid * NC + cid
    base = wid * BPW
    iota = lax.iota(jnp.int32, NLANE)

    pltpu.sync_copy(title_hbm.at[pl.ds(base, BPW)], t2d)
    pltpu.sync_copy(snip_hbm.at[pl.ds(base, BPW)], s2d)

    def body(r, carry):
        _flatten_row(t2d, tfl, r, r * TLEN, TLEN, iota)
        _flatten_row(s2d, sfl, r, r * SLEN, SLEN, iota)
        return carry

    lax.fori_loop(0, BPW, body, 0)

    pltpu.sync_copy(tfl, tout_hbm.at[pl.ds(base * TLEN, BPW * TLEN)])
    pltpu.sync_copy(sfl, sout_hbm.at[pl.ds(base * SLEN, BPW * SLEN)])


def _flatten(title, snippet):
    mesh = plsc.VectorSubcoreMesh(core_axis_name="c", subcore_axis_name="s")
    fn = pl.kernel(
        _flatten_body,
        mesh=mesh,
        out_type=(jax.ShapeDtypeStruct((B * TLEN,), jnp.int32),
                  jax.ShapeDtypeStruct((B * SLEN,), jnp.int32)),
        scratch_types=[
            pltpu.VMEM((BPW, TLEN), jnp.int32),
            pltpu.VMEM((BPW, SLEN), jnp.int32),
            pltpu.VMEM((BPW * TLEN,), jnp.int32),
            pltpu.VMEM((BPW * SLEN,), jnp.int32),
        ],
        compiler_params=pltpu.CompilerParams(needs_layout_passes=False),
    )
    return fn(title, snippet)


TG = GROUP * TLEN
SG = GROUP * SLEN


def _pool_body(tflat_hbm, sflat_hbm, ttab_hbm, stab_hbm, out_hbm,
               idx_t, idx_s, tbufA, sbufA, tbufB, sbufB, acc, semA, semB):
    cid = lax.axis_index("c")
    sid = lax.axis_index("s")
    wid = sid * NC + cid
    base = wid * BPW

    pltpu.sync_copy(tflat_hbm.at[pl.ds(base * TLEN, BPW * TLEN)], idx_t)
    pltpu.sync_copy(sflat_hbm.at[pl.ds(base * SLEN, BPW * SLEN)], idx_s)

    def fire(g, tbuf, sbuf, sem):
        pltpu.async_copy(ttab_hbm.at[idx_t.at[pl.ds(g * TG, TG)]], tbuf, sem)
        pltpu.async_copy(stab_hbm.at[idx_s.at[pl.ds(g * SG, SG)]], sbuf, sem)

    def drain(g, tbuf, sbuf, sem):
        pltpu.make_async_copy(
            ttab_hbm.at[idx_t.at[pl.ds(g * TG, TG)]], tbuf, sem).wait()
        pltpu.make_async_copy(
            stab_hbm.at[idx_s.at[pl.ds(g * SG, SG)]], sbuf, sem).wait()

    def accumulate(g, tbuf, sbuf):
        zeros = tuple(jnp.zeros((NLANE,), jnp.float32) for _ in range(NV))
        for k in range(GROUP):
            row = g * GROUP + k

            def tbody(t, vs):
                b = k * TLEN + 2 * t
                return tuple(
                    vs[c] + tbuf[b, pl.ds(c * NLANE, NLANE)]
                    + tbuf[b + 1, pl.ds(c * NLANE, NLANE)]
                    for c in range(NV))

            tv = lax.fori_loop(0, TLEN // 2, tbody, zeros)
            for c in range(NV):
                acc[row, pl.ds(c * NLANE, NLANE)] = tv[c] * (1.0 / TLEN)

            def sbody(t, vs):
                b = k * SLEN + 2 * t
                return tuple(
                    vs[c] + sbuf[b, pl.ds(c * NLANE, NLANE)]
                    + sbuf[b + 1, pl.ds(c * NLANE, NLANE)]
                    for c in range(NV))

            sv = lax.fori_loop(0, SLEN // 2, sbody, zeros)
            for c in range(NV):
                acc[row, pl.ds(D + c * NLANE, NLANE)] = sv[c] * (1.0 / SLEN)

    fire(0, tbufA, sbufA, semA)
    bufs = ((tbufA, sbufA, semA), (tbufB, sbufB, semB))

    def body(i, carry):
        for p in range(2):
            g = i * 2 + p
            tbuf, sbuf, sem = bufs[p]
            ntbuf, nsbuf, nsem = bufs[1 - p]

            @pl.when(g + 1 < NG)
            def _():
                fire(g + 1, ntbuf, nsbuf, nsem)

            drain(g, tbuf, sbuf, sem)
            accumulate(g, tbuf, sbuf)
        return carry

    lax.fori_loop(0, NG // 2, body, 0)

    pltpu.sync_copy(acc, out_hbm.at[pl.ds(base, BPW)])


def _pool(tflat, sflat, ttab, stab):
    mesh = plsc.VectorSubcoreMesh(core_axis_name="c", subcore_axis_name="s")
    fn = pl.kernel(
        _pool_body,
        mesh=mesh,
        out_type=jax.ShapeDtypeStruct((B, 2 * D), jnp.float32),
        scratch_types=[
            pltpu.VMEM((BPW * TLEN,), jnp.int32),
            pltpu.VMEM((BPW * SLEN,), jnp.int32),
            pltpu.VMEM((TG, D), jnp.float32),
            pltpu.VMEM((SG, D), jnp.float32),
            pltpu.VMEM((TG, D), jnp.float32),
            pltpu.VMEM((SG, D), jnp.float32),
            pltpu.VMEM((BPW, 2 * D), jnp.float32),
            pltpu.SemaphoreType.DMA,
            pltpu.SemaphoreType.DMA,
        ],
        compiler_params=pltpu.CompilerParams(use_tc_tiling_on_sc=False),
    )
    return fn(tflat, sflat, ttab, stab)


def _mlp_body(x_ref, w1_ref, b1_ref, w2_ref, b2_ref, o_ref):
    h = jnp.dot(x_ref[...], w1_ref[...], preferred_element_type=jnp.float32)
    h = jnp.maximum(h + b1_ref[...], 0.0)
    o_ref[...] = (jnp.dot(h, w2_ref[...], preferred_element_type=jnp.float32)
                  + b2_ref[...])


def _mlp(x, W1, b1, W2, b2):
    TB = 512
    grid = (B // TB,)
    return pl.pallas_call(
        _mlp_body,
        grid=grid,
        in_specs=[
            pl.BlockSpec((TB, 2 * D), lambda i: (i, 0)),
            pl.BlockSpec((2 * D, HID), lambda i: (0, 0)),
            pl.BlockSpec((1, HID), lambda i: (0, 0)),
            pl.BlockSpec((HID, OUT), lambda i: (0, 0)),
            pl.BlockSpec((1, OUT), lambda i: (0, 0)),
        ],
        out_specs=pl.BlockSpec((TB, OUT), lambda i: (i, 0)),
        out_shape=jax.ShapeDtypeStruct((B, OUT), jnp.float32),
    )(x, W1, b1, W2, b2)


def kernel(title, snippet, title_table, snippet_table, W1, b1, W2, b2):
    tflat, sflat = _flatten(title.astype(jnp.int32), snippet.astype(jnp.int32))
    combined = _pool(tflat, sflat, title_table, snippet_table)
    return _mlp(combined, W1, b1.reshape(1, HID), W2, b2.reshape(1, OUT))
